# baseline (speedup 1.0000x reference)
import jax
import jax.numpy as jnp
from jax.experimental import pallas as pl
from jax.experimental.pallas import tpu as pltpu

_SEQ = 2048
_SLAB = 16


def _update_kernel(pos_ref, k_val, v_val, kw_val, k_in, v_in, kw_in,
                   k_out, v_out, kw_out,
                   blk_k, blk_v, kv_buf, vv_buf,
                   sk, sv, skw, s_in, s_val, s_wr, s_kwp):
    pos = pos_ref[0] % _SEQ
    base = pl.multiple_of((pos // _SLAB) * _SLAB, _SLAB)
    off = pos - base

    ck = pltpu.make_async_copy(k_in, k_out, sk)
    cv = pltpu.make_async_copy(v_in, v_out, sv)
    ckw = pltpu.make_async_copy(kw_in, kw_out, skw)
    ck.start()
    cv.start()
    ckw.start()

    rk = pltpu.make_async_copy(k_in.at[:, pl.ds(base, _SLAB), :], blk_k, s_in)
    rv = pltpu.make_async_copy(v_in.at[:, pl.ds(base, _SLAB), :], blk_v, s_in)
    gk = pltpu.make_async_copy(k_val, kv_buf, s_val)
    gv = pltpu.make_async_copy(v_val, vv_buf, s_val)
    rk.start()
    rv.start()
    gk.start()
    gv.start()
    rk.wait()
    rv.wait()
    gk.wait()
    gv.wait()

    ids = jax.lax.broadcasted_iota(jnp.int32, blk_k.shape, 1)
    sel = ids == off
    blk_k[...] = jnp.where(sel, kv_buf[...], blk_k[...])
    blk_v[...] = jnp.where(sel, vv_buf[...], blk_v[...])

    ck.wait()
    wk = pltpu.make_async_copy(blk_k, k_out.at[:, pl.ds(base, _SLAB), :], s_wr)
    wk.start()
    cv.wait()
    wv = pltpu.make_async_copy(blk_v, v_out.at[:, pl.ds(base, _SLAB), :], s_wr)
    wv.start()
    ckw.wait()
    pkw = pltpu.make_async_copy(kw_val, kw_out.at[:, pl.ds(pos, 1), :, :, :], s_kwp)
    pkw.start()
    wk.wait()
    wv.wait()
    pkw.wait()


def kernel(input_pos, k_val, v_val, kw_val, k_cache, v_cache, kw_cache):
    B, N, S, D = k_cache.shape
    bc = lambda x: jax.lax.bitcast_convert_type(x, jnp.bfloat16)
    k_in3 = bc(k_cache).reshape(B * N, S, D)
    v_in3 = bc(v_cache).reshape(B * N, S, D)
    kv3 = jnp.broadcast_to(bc(k_val).reshape(B * N, 1, D), (B * N, _SLAB, D))
    vv3 = jnp.broadcast_to(bc(v_val).reshape(B * N, 1, D), (B * N, _SLAB, D))
    kw_in = bc(kw_val)
    kw_c = bc(kw_cache)
    out_shape = (
        jax.ShapeDtypeStruct(k_in3.shape, jnp.bfloat16),
        jax.ShapeDtypeStruct(v_in3.shape, jnp.bfloat16),
        jax.ShapeDtypeStruct(kw_cache.shape, jnp.bfloat16),
    )
    hbm = pl.BlockSpec(memory_space=pltpu.MemorySpace.HBM)
    k_out, v_out, kw_out = pl.pallas_call(
        _update_kernel,
        out_shape=out_shape,
        in_specs=[
            pl.BlockSpec(memory_space=pltpu.MemorySpace.SMEM),
            hbm, hbm, hbm, hbm, hbm, hbm,
        ],
        out_specs=(hbm, hbm, hbm),
        scratch_shapes=[
            pltpu.VMEM((B * N, _SLAB, D), jnp.bfloat16),
            pltpu.VMEM((B * N, _SLAB, D), jnp.bfloat16),
            pltpu.VMEM((B * N, _SLAB, D), jnp.bfloat16),
            pltpu.VMEM((B * N, _SLAB, D), jnp.bfloat16),
        ] + [pltpu.SemaphoreType.DMA] * 7,
    )(input_pos.astype(jnp.int32), kv3, vv3, kw_in,
      k_in3, v_in3, kw_c)
    ic = lambda x: jax.lax.bitcast_convert_type(x, jnp.float16)
    return (ic(k_out).reshape(B, N, S, D), ic(v_out).reshape(B, N, S, D),
            ic(kw_out))


# speedup vs baseline: 13.5075x; 13.5075x over previous
import jax
import jax.numpy as jnp
from jax.experimental import pallas as pl
from jax.experimental.pallas import tpu as pltpu

_SEQ = 2048
_BS = 32
_GRID = _SEQ // _BS


def _update_kernel(pos_ref, kval, vval, kwval, k_in, v_in, kw_in,
                   k_out, v_out, kw_out):
    g = pl.program_id(0)
    pos = pos_ref[0] % _SEQ
    base = g * _BS

    ids = base + jax.lax.broadcasted_iota(jnp.int32, k_in.shape, 1)
    sel = ids == pos
    k_out[...] = jnp.where(sel, kval[...], k_in[...])
    v_out[...] = jnp.where(sel, vval[...], v_in[...])

    ids_kw = base + jax.lax.broadcasted_iota(jnp.int32, kw_in.shape, 1)
    kw_out[...] = jnp.where(ids_kw == pos, kwval[...], kw_in[...])


def kernel(input_pos, k_val, v_val, kw_val, k_cache, v_cache, kw_cache):
    B, N, S, D = k_cache.shape
    bc = lambda x: jax.lax.bitcast_convert_type(x, jnp.bfloat16)
    k_in3 = bc(k_cache).reshape(B * N, S, D)
    v_in3 = bc(v_cache).reshape(B * N, S, D)
    kv3 = jnp.broadcast_to(bc(k_val).reshape(B * N, 1, D), (B * N, _BS, D))
    vv3 = jnp.broadcast_to(bc(v_val).reshape(B * N, 1, D), (B * N, _BS, D))
    kwv = jnp.broadcast_to(bc(kw_val), (B, _BS, 2, N, N))
    kw_in = bc(kw_cache)

    out_shape = (
        jax.ShapeDtypeStruct(k_in3.shape, jnp.bfloat16),
        jax.ShapeDtypeStruct(v_in3.shape, jnp.bfloat16),
        jax.ShapeDtypeStruct(kw_in.shape, jnp.bfloat16),
    )
    kv_spec = pl.BlockSpec((B * N, _BS, D), lambda g: (0, g, 0))
    kwc_spec = pl.BlockSpec((B, _BS, 2, N, N), lambda g: (0, g, 0, 0, 0))
    k_out, v_out, kw_out = pl.pallas_call(
        _update_kernel,
        grid=(_GRID,),
        out_shape=out_shape,
        in_specs=[
            pl.BlockSpec(memory_space=pltpu.MemorySpace.SMEM),
            pl.BlockSpec((B * N, _BS, D), lambda g: (0, 0, 0)),
            pl.BlockSpec((B * N, _BS, D), lambda g: (0, 0, 0)),
            pl.BlockSpec((B, _BS, 2, N, N), lambda g: (0, 0, 0, 0, 0)),
            kv_spec, kv_spec, kwc_spec,
        ],
        out_specs=(kv_spec, kv_spec, kwc_spec),
        compiler_params=pltpu.CompilerParams(
            dimension_semantics=("arbitrary",),
        ),
    )(input_pos.astype(jnp.int32), kv3, vv3, kwv, k_in3, v_in3, kw_in)
    ic = lambda x: jax.lax.bitcast_convert_type(x, jnp.float16)
    return (ic(k_out).reshape(B, N, S, D), ic(v_out).reshape(B, N, S, D),
            ic(kw_out))
